# baseline (speedup 1.0000x reference)
import functools

import jax
import jax.numpy as jnp
from jax import lax
from jax.experimental import pallas as pl
from jax.experimental.pallas import tpu as pltpu
from jax.experimental.pallas import tpu_sc as plsc

EMB = 64
HID = 256
TAGS = 20
B = 4096
L = 200
HA = 128
HB = 72
LANES = 16
NC = 2
NS = 16
NW = NC * NS
SPW = B // NW


LP = 256


EMBP = 128


def _embedding_bag_sc(xp, embp):
    mesh = plsc.VectorSubcoreMesh(core_axis_name="c", subcore_axis_name="s")

    @functools.partial(
        pl.kernel,
        out_type=jax.ShapeDtypeStruct((B, EMB), jnp.float32),
        mesh=mesh,
        compiler_params=pltpu.CompilerParams(use_tc_tiling_on_sc=True),
        scratch_types=[
            pltpu.VMEM((SPW, LP), jnp.int32),
            pltpu.VMEM((2, L, EMBP), jnp.float32),
            pltpu.VMEM((SPW, EMB), jnp.float32),
            pltpu.SemaphoreType.DMA,
            pltpu.SemaphoreType.DMA,
        ],
    )
    def bag(x_hbm, emb_hbm, out_hbm, idx_v, rows_v, out_v, sem0, sem1):
        wid = lax.axis_index("s") * NC + lax.axis_index("c")
        base = wid * SPW
        pltpu.sync_copy(x_hbm.at[pl.ds(base, SPW)], idx_v)
        sems = (sem0, sem1)

        def issue(s, b):
            pltpu.async_copy(
                emb_hbm.at[idx_v.at[s, pl.ds(0, HA)]],
                rows_v.at[b, pl.ds(0, HA)], sems[b])
            pltpu.async_copy(
                emb_hbm.at[idx_v.at[s, pl.ds(HA, HB)]],
                rows_v.at[b, pl.ds(HA, HB)], sems[b])

        def drain(b):
            pltpu.make_async_copy(
                emb_hbm.at[pl.ds(0, L)], rows_v.at[b], sems[b]).wait()

        def reduce_store(s, b):
            def rbody(i, accs):
                r = i * 8
                out = []
                for k in range(4):
                    v = [rows_v[b, r + j, pl.ds(LANES * k, LANES)]
                         for j in range(8)]
                    t = ((v[0] + v[1]) + (v[2] + v[3])) \
                        + ((v[4] + v[5]) + (v[6] + v[7]))
                    out.append(accs[k] + t)
                return tuple(out)

            zero = jnp.zeros((LANES,), jnp.float32)
            accs = lax.fori_loop(0, L // 8, rbody, (zero,) * 4)
            for k in range(4):
                out_v[s, pl.ds(LANES * k, LANES)] = accs[k]

        issue(0, 0)

        def body2(i, carry):
            s0 = 2 * i
            issue(s0 + 1, 1)
            drain(0)
            reduce_store(s0, 0)

            @pl.when(i + 1 < SPW // 2)
            def _():
                issue(s0 + 2, 0)

            drain(1)
            reduce_store(s0 + 1, 1)
            return carry

        lax.fori_loop(0, SPW // 2, body2, 0)
        pltpu.sync_copy(out_v, out_hbm.at[pl.ds(base, SPW)])

    return bag(xp, embp)


def _transpose_table_sc(embTp):
    VP = embTp.shape[1]
    NCH = VP // 128
    CPW = (NCH + NW - 1) // NW
    mesh = plsc.VectorSubcoreMesh(core_axis_name="c", subcore_axis_name="s")

    @functools.partial(
        pl.kernel,
        out_type=jax.ShapeDtypeStruct((VP, EMBP), jnp.float32),
        mesh=mesh,
        compiler_params=pltpu.CompilerParams(
            use_tc_tiling_on_sc=True, needs_layout_passes=False),
        scratch_types=[
            pltpu.VMEM((2, EMB, 128), jnp.float32),
            pltpu.VMEM((2, 128, EMBP), jnp.float32),
            pltpu.SemaphoreType.DMA,
            pltpu.SemaphoreType.DMA,
            pltpu.SemaphoreType.DMA,
            pltpu.SemaphoreType.DMA,
        ],
    )
    def tr(in_hbm, out_hbm, in_v, out_v, si0, si1, so0, so1):
        wid = lax.axis_index("s") * NC + lax.axis_index("c")
        sis = (si0, si1)
        sos = (so0, so1)
        col16 = [jax.lax.iota(jnp.int32, 16) + 16 * k for k in range(4)]

        def chunk_of(k):
            return (wid + k * NW) * 128

        def stage(k, b):
            @pl.when(chunk_of(k) < VP)
            def _():
                pltpu.async_copy(
                    in_hbm.at[pl.ds(0, EMB), pl.ds(chunk_of(k), 128)],
                    in_v.at[b], sis[b])

        def drain_in(b):
            pltpu.make_async_copy(
                in_hbm.at[pl.ds(0, EMB), pl.ds(0, 128)],
                in_v.at[b], sis[b]).wait()

        def transpose(b):
            def vbody(v, carry):
                vsplat = jnp.full((16,), v, jnp.int32)
                bsplat = jnp.full((16,), b, jnp.int32)
                for k in range(4):
                    vals = plsc.load_gather(
                        in_v, [bsplat, col16[k], vsplat])
                    out_v[b, v, pl.ds(16 * k, 16)] = vals
                return carry

            lax.fori_loop(0, 128, vbody, 0)

        def store(k, b):
            @pl.when(chunk_of(k) < VP)
            def _():
                pltpu.async_copy(
                    out_v.at[b], out_hbm.at[pl.ds(chunk_of(k), 128)], sos[b])

        def drain_out(b):
            pltpu.make_async_copy(
                out_v.at[b], out_hbm.at[pl.ds(0, 128)], sos[b]).wait()

        stage(0, 0)

        def body2(j, carry):
            k0 = 2 * j
            stage(k0 + 1, 1)

            @pl.when(chunk_of(k0) < VP)
            def _():
                drain_in(0)

                @pl.when(k0 >= 2)
                def _():
                    drain_out(0)
                transpose(0)
                store(k0, 0)

            stage(k0 + 2, 0)

            @pl.when(chunk_of(k0 + 1) < VP)
            def _():
                drain_in(1)

                @pl.when(k0 + 1 >= 2)
                def _():
                    drain_out(1)
                transpose(1)
                store(k0 + 1, 1)

            return carry

        lax.fori_loop(0, (CPW + 1) // 2, body2, 0)
        drain_out(0)
        drain_out(1)

    return tr(embTp)


def _pad_table_tc(embT):
    V = embT.shape[1]
    blkv = 2048

    def body(t_ref, o_ref):
        o_ref[:, 0:EMB] = t_ref[...].T

    return pl.pallas_call(
        body,
        grid=(pl.cdiv(V, blkv),),
        in_specs=[pl.BlockSpec((EMB, blkv), lambda i: (0, i))],
        out_specs=pl.BlockSpec((blkv, EMBP), lambda i: (i, 0)),
        out_shape=jax.ShapeDtypeStruct((V, EMBP), jnp.float32),
    )(embT)


def _mlp_tc(sums, W1, b1, W2, b2):
    blk = 512

    def mlp_body(s_ref, w1_ref, b1_ref, w2_ref, b2_ref, o_ref):
        xa = s_ref[...] * (1.0 / L)
        h = jnp.dot(xa, w1_ref[...], preferred_element_type=jnp.float32)
        h = jnp.maximum(h + b1_ref[...], 0.0)
        o_ref[...] = (
            jnp.dot(h, w2_ref[...], preferred_element_type=jnp.float32)
            + b2_ref[...])

    return pl.pallas_call(
        mlp_body,
        grid=(B // blk,),
        in_specs=[
            pl.BlockSpec((blk, EMB), lambda i: (i, 0)),
            pl.BlockSpec((EMB, HID), lambda i: (0, 0)),
            pl.BlockSpec((1, HID), lambda i: (0, 0)),
            pl.BlockSpec((HID, TAGS), lambda i: (0, 0)),
            pl.BlockSpec((1, TAGS), lambda i: (0, 0)),
        ],
        out_specs=pl.BlockSpec((blk, TAGS), lambda i: (i, 0)),
        out_shape=jax.ShapeDtypeStruct((B, TAGS), jnp.float32),
    )(sums, W1, b1, W2, b2)


def kernel(x, emb, W1, b1, W2, b2):
    xp = jnp.pad(x, ((0, 0), (0, LP - L)))
    embT = emb.T
    vpad = (-embT.shape[1]) % 128
    embp = _transpose_table_sc(jnp.pad(embT, ((0, 0), (0, vpad))))
    sums = _embedding_bag_sc(xp, embp)
    return _mlp_tc(sums, W1, b1.reshape(1, HID), W2, b2.reshape(1, TAGS))


# speedup vs baseline: 3.8226x; 3.8226x over previous
import functools

import jax
import jax.numpy as jnp
from jax import lax
from jax.experimental import pallas as pl
from jax.experimental.pallas import tpu as pltpu
from jax.experimental.pallas import tpu_sc as plsc

EMB = 64
HID = 256
TAGS = 20
B = 4096
L = 200
HA = 128
HB = 72
LANES = 16
NC = 2
NS = 16
NW = NC * NS
SPW = B // NW


LP = 256


EMBP = 128


def _embedding_bag_sc(xp, embp):
    mesh = plsc.VectorSubcoreMesh(core_axis_name="c", subcore_axis_name="s")

    @functools.partial(
        pl.kernel,
        out_type=jax.ShapeDtypeStruct((B, EMB), jnp.float32),
        mesh=mesh,
        compiler_params=pltpu.CompilerParams(use_tc_tiling_on_sc=True),
        scratch_types=[
            pltpu.VMEM((SPW, LP), jnp.int32),
            pltpu.VMEM((2, L, EMBP), jnp.float32),
            pltpu.VMEM((SPW, EMB), jnp.float32),
            pltpu.SemaphoreType.DMA,
            pltpu.SemaphoreType.DMA,
        ],
    )
    def bag(x_hbm, emb_hbm, out_hbm, idx_v, rows_v, out_v, sem0, sem1):
        wid = lax.axis_index("s") * NC + lax.axis_index("c")
        base = wid * SPW
        pltpu.sync_copy(x_hbm.at[pl.ds(base, SPW)], idx_v)
        sems = (sem0, sem1)

        def issue(s, b):
            pltpu.async_copy(
                emb_hbm.at[idx_v.at[s, pl.ds(0, HA)]],
                rows_v.at[b, pl.ds(0, HA)], sems[b])
            pltpu.async_copy(
                emb_hbm.at[idx_v.at[s, pl.ds(HA, HB)]],
                rows_v.at[b, pl.ds(HA, HB)], sems[b])

        def drain(b):
            pltpu.make_async_copy(
                emb_hbm.at[pl.ds(0, L)], rows_v.at[b], sems[b]).wait()

        def reduce_store(s, b):
            def rbody(i, accs):
                r = i * 8
                out = []
                for k in range(4):
                    v = [rows_v[b, r + j, pl.ds(LANES * k, LANES)]
                         for j in range(8)]
                    t = ((v[0] + v[1]) + (v[2] + v[3])) \
                        + ((v[4] + v[5]) + (v[6] + v[7]))
                    out.append(accs[k] + t)
                return tuple(out)

            zero = jnp.zeros((LANES,), jnp.float32)
            accs = lax.fori_loop(0, L // 8, rbody, (zero,) * 4)
            for k in range(4):
                out_v[s, pl.ds(LANES * k, LANES)] = accs[k]

        issue(0, 0)

        def body2(i, carry):
            s0 = 2 * i
            issue(s0 + 1, 1)
            drain(0)
            reduce_store(s0, 0)

            @pl.when(i + 1 < SPW // 2)
            def _():
                issue(s0 + 2, 0)

            drain(1)
            reduce_store(s0 + 1, 1)
            return carry

        lax.fori_loop(0, SPW // 2, body2, 0)
        pltpu.sync_copy(out_v, out_hbm.at[pl.ds(base, SPW)])

    return bag(xp, embp)


def _pad_table_tc(embT):
    V = embT.shape[1]
    blkv = 8192

    def body(t_ref, o_ref):
        o_ref[:, 0:EMB] = t_ref[...].T

    return pl.pallas_call(
        body,
        grid=(pl.cdiv(V, blkv),),
        in_specs=[pl.BlockSpec((EMB, blkv), lambda i: (0, i))],
        out_specs=pl.BlockSpec((blkv, EMBP), lambda i: (i, 0)),
        out_shape=jax.ShapeDtypeStruct((V, EMBP), jnp.float32),
    )(embT)


def _mlp_tc(sums, W1, b1, W2, b2):
    blk = 512

    def mlp_body(s_ref, w1_ref, b1_ref, w2_ref, b2_ref, o_ref):
        xa = s_ref[...] * (1.0 / L)
        h = jnp.dot(xa, w1_ref[...], preferred_element_type=jnp.float32)
        h = jnp.maximum(h + b1_ref[...], 0.0)
        o_ref[...] = (
            jnp.dot(h, w2_ref[...], preferred_element_type=jnp.float32)
            + b2_ref[...])

    return pl.pallas_call(
        mlp_body,
        grid=(B // blk,),
        in_specs=[
            pl.BlockSpec((blk, EMB), lambda i: (i, 0)),
            pl.BlockSpec((EMB, HID), lambda i: (0, 0)),
            pl.BlockSpec((1, HID), lambda i: (0, 0)),
            pl.BlockSpec((HID, TAGS), lambda i: (0, 0)),
            pl.BlockSpec((1, TAGS), lambda i: (0, 0)),
        ],
        out_specs=pl.BlockSpec((blk, TAGS), lambda i: (i, 0)),
        out_shape=jax.ShapeDtypeStruct((B, TAGS), jnp.float32),
    )(sums, W1, b1, W2, b2)


def kernel(x, emb, W1, b1, W2, b2):
    xp = jnp.pad(x, ((0, 0), (0, LP - L)))
    embp = _pad_table_tc(emb.T)
    sums = _embedding_bag_sc(xp, embp)
    return _mlp_tc(sums, W1, b1.reshape(1, HID), W2, b2.reshape(1, TAGS))


# speedup vs baseline: 3.9806x; 1.0413x over previous
import functools

import jax
import jax.numpy as jnp
from jax import lax
from jax.experimental import pallas as pl
from jax.experimental.pallas import tpu as pltpu
from jax.experimental.pallas import tpu_sc as plsc

EMB = 64
HID = 256
TAGS = 20
B = 4096
L = 200
HA = 128
HB = 72
LANES = 16
NC = 2
NS = 16
NW = NC * NS
SPW = B // NW


LP = 256


EMBP = 128


def _embedding_bag_sc(xp, embp):
    mesh = plsc.VectorSubcoreMesh(core_axis_name="c", subcore_axis_name="s")

    @functools.partial(
        pl.kernel,
        out_type=jax.ShapeDtypeStruct((B, EMB), jnp.float32),
        mesh=mesh,
        compiler_params=pltpu.CompilerParams(use_tc_tiling_on_sc=True),
        scratch_types=[
            pltpu.VMEM((SPW, LP), jnp.int32),
            pltpu.VMEM((2, L, EMBP), jnp.float32),
            pltpu.VMEM((SPW, EMB), jnp.float32),
            pltpu.SemaphoreType.DMA,
            pltpu.SemaphoreType.DMA,
        ],
    )
    def bag(x_hbm, emb_hbm, out_hbm, idx_v, rows_v, out_v, sem0, sem1):
        wid = lax.axis_index("s") * NC + lax.axis_index("c")
        base = wid * SPW
        pltpu.sync_copy(x_hbm.at[pl.ds(base, SPW)], idx_v)
        sems = (sem0, sem1)

        def issue(s, b):
            pltpu.async_copy(
                emb_hbm.at[idx_v.at[s, pl.ds(0, HA)]],
                rows_v.at[b, pl.ds(0, HA)], sems[b])
            pltpu.async_copy(
                emb_hbm.at[idx_v.at[s, pl.ds(HA, HB)]],
                rows_v.at[b, pl.ds(HA, HB)], sems[b])

        def drain(b):
            pltpu.make_async_copy(
                emb_hbm.at[pl.ds(0, L)], rows_v.at[b], sems[b]).wait()

        def reduce_store(s, b):
            def rbody(i, accs):
                r = i * 8
                out = []
                for k in range(4):
                    v = [rows_v[b, r + j, pl.ds(LANES * k, LANES)]
                         for j in range(8)]
                    t = ((v[0] + v[1]) + (v[2] + v[3])) \
                        + ((v[4] + v[5]) + (v[6] + v[7]))
                    out.append(accs[k] + t)
                return tuple(out)

            zero = jnp.zeros((LANES,), jnp.float32)
            accs = lax.fori_loop(0, L // 8, rbody, (zero,) * 4)
            for k in range(4):
                out_v[s, pl.ds(LANES * k, LANES)] = accs[k]

        issue(0, 0)

        def body2(i, carry):
            s0 = 2 * i
            issue(s0 + 1, 1)
            drain(0)
            reduce_store(s0, 0)

            @pl.when(i + 1 < SPW // 2)
            def _():
                issue(s0 + 2, 0)

            drain(1)
            reduce_store(s0 + 1, 1)
            return carry

        lax.fori_loop(0, SPW // 2, body2, 0)
        pltpu.sync_copy(out_v, out_hbm.at[pl.ds(base, SPW)])

    return bag(xp, embp)


def _pad_table_tc(embT):
    V = embT.shape[1]
    blkv = 16384

    def body(t_ref, o_ref):
        o_ref[:, 0:EMB] = t_ref[...].T

    return pl.pallas_call(
        body,
        grid=(pl.cdiv(V, blkv),),
        in_specs=[pl.BlockSpec((EMB, blkv), lambda i: (0, i))],
        out_specs=pl.BlockSpec((blkv, EMBP), lambda i: (i, 0)),
        out_shape=jax.ShapeDtypeStruct((V, EMBP), jnp.float32),
    )(embT)


def _mlp_tc(sums, W1, b1, W2, b2):
    blk = 512

    def mlp_body(s_ref, w1_ref, b1_ref, w2_ref, b2_ref, o_ref):
        xa = s_ref[...] * (1.0 / L)
        h = jnp.dot(xa, w1_ref[...], preferred_element_type=jnp.float32)
        h = jnp.maximum(h + b1_ref[...], 0.0)
        o_ref[...] = (
            jnp.dot(h, w2_ref[...], preferred_element_type=jnp.float32)
            + b2_ref[...])

    return pl.pallas_call(
        mlp_body,
        grid=(B // blk,),
        in_specs=[
            pl.BlockSpec((blk, EMB), lambda i: (i, 0)),
            pl.BlockSpec((EMB, HID), lambda i: (0, 0)),
            pl.BlockSpec((1, HID), lambda i: (0, 0)),
            pl.BlockSpec((HID, TAGS), lambda i: (0, 0)),
            pl.BlockSpec((1, TAGS), lambda i: (0, 0)),
        ],
        out_specs=pl.BlockSpec((blk, TAGS), lambda i: (i, 0)),
        out_shape=jax.ShapeDtypeStruct((B, TAGS), jnp.float32),
    )(sums, W1, b1, W2, b2)


def kernel(x, emb, W1, b1, W2, b2):
    xp = jnp.pad(x, ((0, 0), (0, LP - L)))
    embp = _pad_table_tc(emb.T)
    sums = _embedding_bag_sc(xp, embp)
    return _mlp_tc(sums, W1, b1.reshape(1, HID), W2, b2.reshape(1, TAGS))
